# baseline (speedup 1.0000x reference)
import jax
import jax.numpy as jnp
from jax.experimental import pallas as pl
from jax.experimental.pallas import tpu as pltpu


def _lstm_kernel(x_ref, w_ref, b_ref, wfc_ref, bfc_ref,
                 out_ref, h_sc, c_sc):
    chunk = pl.program_id(1)
    n_chunks = pl.num_programs(1)
    tc, Bt, D = x_ref.shape
    H = h_sc.shape[1]
    Bh = Bt // 2

    @pl.when(chunk == 0)
    def _init():
        h_sc[...] = jnp.zeros_like(h_sc)
        c_sc[...] = jnp.zeros_like(c_sc)

    def _half(t, lo, hi, h_bf, c):
        z = jnp.concatenate([x_ref[t, lo:hi, :], h_bf], axis=1)
        gates = (jnp.dot(z, w_ref[...], preferred_element_type=jnp.float32)
                 + b_ref[...])
        sig = 0.5 * jnp.tanh(gates[:, :3 * H]) + 0.5
        i = sig[:, 0 * H:1 * H]
        f = sig[:, 1 * H:2 * H]
        o = sig[:, 2 * H:3 * H]
        g = jnp.tanh(gates[:, 3 * H:])
        c_new = f * c + i * g
        h_new = o * jnp.tanh(c_new)
        return h_new.astype(jnp.bfloat16), c_new

    ha = h_sc[:Bh, :].astype(jnp.bfloat16)
    hb = h_sc[Bh:, :].astype(jnp.bfloat16)
    ca = c_sc[:Bh, :]
    cb = c_sc[Bh:, :]
    for t in range(tc):
        ha, ca = _half(t, 0, Bh, ha, ca)
        hb, cb = _half(t, Bh, Bt, hb, cb)
    c_sc[:Bh, :] = ca
    c_sc[Bh:, :] = cb
    h_sc[:Bh, :] = ha.astype(jnp.float32)
    h_sc[Bh:, :] = hb.astype(jnp.float32)

    @pl.when(chunk == n_chunks - 1)
    def _fc():
        out_ref[...] = (jnp.dot(h_sc[...].astype(wfc_ref.dtype), wfc_ref[...],
                                preferred_element_type=jnp.float32)
                        + bfc_ref[...]).astype(out_ref.dtype)


def _permute_ifgo_to_ifog(w):
    i, f, g, o = jnp.split(w, 4, axis=0)
    return jnp.concatenate([i, f, o, g], axis=0)


def _scale_sigmoid_cols(w, H):
    return jnp.concatenate([0.5 * w[:3 * H], w[3 * H:]], axis=0)


def kernel(x, w_ih, w_hh, b_ih, b_hh, w_fc, b_fc):
    B, T, D = x.shape
    H = w_hh.shape[1]
    C = w_fc.shape[0]
    G = 4 * H

    b_tile = B // 2
    t_chunk = 16
    n_chunks = T // t_chunk
    mm_dtype = jnp.bfloat16

    x_tm = jnp.transpose(x, (1, 0, 2)).astype(mm_dtype)
    wih_p = _scale_sigmoid_cols(_permute_ifgo_to_ifog(w_ih), H)
    whh_p = _scale_sigmoid_cols(_permute_ifgo_to_ifog(w_hh), H)
    b_p = _scale_sigmoid_cols(_permute_ifgo_to_ifog(b_ih + b_hh), H)
    w_fused = jnp.concatenate(
        [jnp.transpose(wih_p), jnp.transpose(whh_p)], axis=0
    ).astype(mm_dtype)
    bias = b_p.reshape(1, G).astype(jnp.float32)
    wfc_t = jnp.transpose(w_fc).astype(mm_dtype)
    bfc = b_fc.reshape(1, C).astype(jnp.float32)

    def _const(shape):
        return pl.BlockSpec(shape, lambda bt, c: (0, 0))

    out = pl.pallas_call(
        _lstm_kernel,
        out_shape=jax.ShapeDtypeStruct((B, C), jnp.float32),
        grid_spec=pltpu.PrefetchScalarGridSpec(
            num_scalar_prefetch=0,
            grid=(2, n_chunks),
            in_specs=[
                pl.BlockSpec((t_chunk, b_tile, D), lambda bt, c: (c, bt, 0)),
                _const((D + H, G)),
                _const((1, G)),
                _const((H, C)),
                _const((1, C)),
            ],
            out_specs=pl.BlockSpec((b_tile, C), lambda bt, c: (bt, 0)),
            scratch_shapes=[
                pltpu.VMEM((b_tile, H), jnp.float32),
                pltpu.VMEM((b_tile, H), jnp.float32),
            ],
        ),
        compiler_params=pltpu.CompilerParams(
            dimension_semantics=("parallel", "arbitrary"),
            vmem_limit_bytes=100 * 1024 * 1024,
        ),
        cost_estimate=pl.CostEstimate(
            flops=2 * T * B * (D + H) * G + 2 * B * H * C,
            transcendentals=5 * T * B * H,
            bytes_accessed=B * T * D * 2 + (D + H) * G * 2 + B * C * 4,
        ),
    )(x_tm, w_fused, bias, wfc_t, bfc)

    return out


# speedup vs baseline: 1.2399x; 1.2399x over previous
import jax
import jax.numpy as jnp
from jax.experimental import pallas as pl
from jax.experimental.pallas import tpu as pltpu


def _lstm_kernel(x_ref, w_ref, b_ref, wfc_ref, bfc_ref,
                 out_ref, h_sc, c_sc):
    chunk = pl.program_id(1)
    n_chunks = pl.num_programs(1)
    tc, Bt, D = x_ref.shape
    H = h_sc.shape[1]
    Bh = Bt // 2

    @pl.when(chunk == 0)
    def _init():
        h_sc[...] = jnp.zeros_like(h_sc)
        c_sc[...] = jnp.zeros_like(c_sc)

    def _step(t, h_bf, c):
        z = jnp.concatenate([x_ref[t], h_bf], axis=1)
        gates = (jnp.dot(z, w_ref[...], preferred_element_type=jnp.float32)
                 + b_ref[...])
        sig = 0.5 * jnp.tanh(gates[:, :3 * H]) + 0.5
        i = sig[:, 0 * H:1 * H]
        f = sig[:, 1 * H:2 * H]
        o = sig[:, 2 * H:3 * H]
        g = jnp.tanh(gates[:, 3 * H:])
        c_new = f * c + i * g
        h_new = o * jnp.tanh(c_new)
        return h_new.astype(jnp.bfloat16), c_new

    h = h_sc[...].astype(jnp.bfloat16)
    c = c_sc[...]
    for t in range(tc):
        h, c = _step(t, h, c)
    c_sc[...] = c
    h_sc[...] = h.astype(jnp.float32)

    @pl.when(chunk == n_chunks - 1)
    def _fc():
        out_ref[...] = (jnp.dot(h_sc[...].astype(wfc_ref.dtype), wfc_ref[...],
                                preferred_element_type=jnp.float32)
                        + bfc_ref[...]).astype(out_ref.dtype)


def _permute_ifgo_to_ifog(w):
    i, f, g, o = jnp.split(w, 4, axis=0)
    return jnp.concatenate([i, f, o, g], axis=0)


def _scale_sigmoid_cols(w, H):
    return jnp.concatenate([0.5 * w[:3 * H], w[3 * H:]], axis=0)


def kernel(x, w_ih, w_hh, b_ih, b_hh, w_fc, b_fc):
    B, T, D = x.shape
    H = w_hh.shape[1]
    C = w_fc.shape[0]
    G = 4 * H

    b_tile = B // 2
    t_chunk = 16
    n_chunks = T // t_chunk
    mm_dtype = jnp.bfloat16

    x_tm = jnp.transpose(x, (1, 0, 2)).astype(mm_dtype)
    wih_p = _scale_sigmoid_cols(_permute_ifgo_to_ifog(w_ih), H)
    whh_p = _scale_sigmoid_cols(_permute_ifgo_to_ifog(w_hh), H)
    b_p = _scale_sigmoid_cols(_permute_ifgo_to_ifog(b_ih + b_hh), H)
    w_fused = jnp.concatenate(
        [jnp.transpose(wih_p), jnp.transpose(whh_p)], axis=0
    ).astype(mm_dtype)
    bias = b_p.reshape(1, G).astype(jnp.float32)
    wfc_t = jnp.transpose(w_fc).astype(mm_dtype)
    bfc = b_fc.reshape(1, C).astype(jnp.float32)

    def _const(shape):
        return pl.BlockSpec(shape, lambda bt, c: (0, 0))

    out = pl.pallas_call(
        _lstm_kernel,
        out_shape=jax.ShapeDtypeStruct((B, C), jnp.float32),
        grid_spec=pltpu.PrefetchScalarGridSpec(
            num_scalar_prefetch=0,
            grid=(2, n_chunks),
            in_specs=[
                pl.BlockSpec((t_chunk, b_tile, D), lambda bt, c: (c, bt, 0)),
                _const((D + H, G)),
                _const((1, G)),
                _const((H, C)),
                _const((1, C)),
            ],
            out_specs=pl.BlockSpec((b_tile, C), lambda bt, c: (bt, 0)),
            scratch_shapes=[
                pltpu.VMEM((b_tile, H), jnp.float32),
                pltpu.VMEM((b_tile, H), jnp.float32),
            ],
        ),
        compiler_params=pltpu.CompilerParams(
            dimension_semantics=("parallel", "arbitrary"),
            vmem_limit_bytes=100 * 1024 * 1024,
        ),
        cost_estimate=pl.CostEstimate(
            flops=2 * T * B * (D + H) * G + 2 * B * H * C,
            transcendentals=5 * T * B * H,
            bytes_accessed=B * T * D * 2 + (D + H) * G * 2 + B * C * 4,
        ),
    )(x_tm, w_fused, bias, wfc_t, bfc)

    return out


# speedup vs baseline: 1.2920x; 1.0420x over previous
import jax
import jax.numpy as jnp
from jax.experimental import pallas as pl
from jax.experimental.pallas import tpu as pltpu


def _lstm_kernel(x_ref, wih_ref, whh_ref, b_ref, wfc_ref, bfc_ref,
                 out_ref, h_sc, c_sc):
    chunk = pl.program_id(1)
    n_chunks = pl.num_programs(1)
    tc, Bt, D = x_ref.shape
    H = h_sc.shape[1]
    Bh = Bt // 2

    @pl.when(chunk == 0)
    def _init():
        h_sc[...] = jnp.zeros_like(h_sc)
        c_sc[...] = jnp.zeros_like(c_sc)

    def _step(t, h_bf, c):
        gx = jnp.dot(x_ref[t], wih_ref[...],
                     preferred_element_type=jnp.float32) + b_ref[...]
        gates = gx + jnp.dot(h_bf, whh_ref[...],
                             preferred_element_type=jnp.float32)
        sig = 0.5 * jnp.tanh(gates[:, :3 * H]) + 0.5
        i = sig[:, 0 * H:1 * H]
        f = sig[:, 1 * H:2 * H]
        o = sig[:, 2 * H:3 * H]
        g = jnp.tanh(gates[:, 3 * H:])
        c_new = f * c + i * g
        h_new = o * jnp.tanh(c_new)
        return h_new.astype(jnp.bfloat16), c_new

    h = h_sc[...].astype(jnp.bfloat16)
    c = c_sc[...]
    for t in range(tc):
        h, c = _step(t, h, c)
    c_sc[...] = c
    h_sc[...] = h.astype(jnp.float32)

    @pl.when(chunk == n_chunks - 1)
    def _fc():
        out_ref[...] = (jnp.dot(h_sc[...].astype(wfc_ref.dtype), wfc_ref[...],
                                preferred_element_type=jnp.float32)
                        + bfc_ref[...]).astype(out_ref.dtype)


def _permute_ifgo_to_ifog(w):
    i, f, g, o = jnp.split(w, 4, axis=0)
    return jnp.concatenate([i, f, o, g], axis=0)


def _scale_sigmoid_cols(w, H):
    return jnp.concatenate([0.5 * w[:3 * H], w[3 * H:]], axis=0)


def kernel(x, w_ih, w_hh, b_ih, b_hh, w_fc, b_fc):
    B, T, D = x.shape
    H = w_hh.shape[1]
    C = w_fc.shape[0]
    G = 4 * H

    b_tile = B // 2
    t_chunk = 16
    n_chunks = T // t_chunk
    mm_dtype = jnp.bfloat16

    x_tm = jnp.transpose(x, (1, 0, 2)).astype(mm_dtype)
    wih_p = _scale_sigmoid_cols(_permute_ifgo_to_ifog(w_ih), H)
    whh_p = _scale_sigmoid_cols(_permute_ifgo_to_ifog(w_hh), H)
    b_p = _scale_sigmoid_cols(_permute_ifgo_to_ifog(b_ih + b_hh), H)
    wih_t = jnp.transpose(wih_p).astype(mm_dtype)
    whh_t = jnp.transpose(whh_p).astype(mm_dtype)
    bias = b_p.reshape(1, G).astype(jnp.float32)
    wfc_t = jnp.transpose(w_fc).astype(mm_dtype)
    bfc = b_fc.reshape(1, C).astype(jnp.float32)

    def _const(shape):
        return pl.BlockSpec(shape, lambda bt, c: (0, 0))

    out = pl.pallas_call(
        _lstm_kernel,
        out_shape=jax.ShapeDtypeStruct((B, C), jnp.float32),
        grid_spec=pltpu.PrefetchScalarGridSpec(
            num_scalar_prefetch=0,
            grid=(2, n_chunks),
            in_specs=[
                pl.BlockSpec((t_chunk, b_tile, D), lambda bt, c: (c, bt, 0)),
                _const((D, G)),
                _const((H, G)),
                _const((1, G)),
                _const((H, C)),
                _const((1, C)),
            ],
            out_specs=pl.BlockSpec((b_tile, C), lambda bt, c: (bt, 0)),
            scratch_shapes=[
                pltpu.VMEM((b_tile, H), jnp.float32),
                pltpu.VMEM((b_tile, H), jnp.float32),
            ],
        ),
        compiler_params=pltpu.CompilerParams(
            dimension_semantics=("parallel", "arbitrary"),
            vmem_limit_bytes=100 * 1024 * 1024,
        ),
        cost_estimate=pl.CostEstimate(
            flops=2 * T * B * (D + H) * G + 2 * B * H * C,
            transcendentals=5 * T * B * H,
            bytes_accessed=B * T * D * 2 + (D + H) * G * 2 + B * C * 4,
        ),
    )(x_tm, wih_t, whh_t, bias, wfc_t, bfc)

    return out


# speedup vs baseline: 1.3064x; 1.0111x over previous
import jax
import jax.numpy as jnp
from jax.experimental import pallas as pl
from jax.experimental.pallas import tpu as pltpu


def _lstm_kernel(x_ref, wih_ref, whh_ref, b_ref, wfc_ref, bfc_ref,
                 out_ref, h_sc, c_sc):
    chunk = pl.program_id(1)
    n_chunks = pl.num_programs(1)
    tc, Bt, D = x_ref.shape
    H = h_sc.shape[1]
    Bh = Bt // 2

    @pl.when(chunk == 0)
    def _init():
        h_sc[...] = jnp.zeros_like(h_sc)
        c_sc[...] = jnp.zeros_like(c_sc)

    def _step(t, h_bf, c):
        gx = jnp.dot(x_ref[t], wih_ref[...],
                     preferred_element_type=jnp.float32) + b_ref[...]
        gates = gx + jnp.dot(h_bf, whh_ref[...],
                             preferred_element_type=jnp.float32)
        sig = 0.5 * jnp.tanh(gates[:, :3 * H]) + 0.5
        i = sig[:, 0 * H:1 * H]
        f = sig[:, 1 * H:2 * H]
        o = sig[:, 2 * H:3 * H]
        g = jnp.tanh(gates[:, 3 * H:])
        c_new = f * c + i * g
        h_new = o * jnp.tanh(c_new)
        return h_new.astype(jnp.bfloat16), c_new

    h = h_sc[...].astype(jnp.bfloat16)
    c = c_sc[...]
    for t in range(tc):
        h, c = _step(t, h, c)
    c_sc[...] = c
    h_sc[...] = h.astype(jnp.float32)

    @pl.when(chunk == n_chunks - 1)
    def _fc():
        out_ref[...] = (jnp.dot(h_sc[...].astype(wfc_ref.dtype), wfc_ref[...],
                                preferred_element_type=jnp.float32)
                        + bfc_ref[...]).astype(out_ref.dtype)


def _permute_ifgo_to_ifog(w):
    i, f, g, o = jnp.split(w, 4, axis=0)
    return jnp.concatenate([i, f, o, g], axis=0)


def _scale_sigmoid_cols(w, H):
    return jnp.concatenate([0.5 * w[:3 * H], w[3 * H:]], axis=0)


def kernel(x, w_ih, w_hh, b_ih, b_hh, w_fc, b_fc):
    B, T, D = x.shape
    H = w_hh.shape[1]
    C = w_fc.shape[0]
    G = 4 * H

    b_tile = B // 2
    t_chunk = 32
    n_chunks = T // t_chunk
    mm_dtype = jnp.bfloat16

    x_tm = jnp.transpose(x, (1, 0, 2)).astype(mm_dtype)
    wih_p = _scale_sigmoid_cols(_permute_ifgo_to_ifog(w_ih), H)
    whh_p = _scale_sigmoid_cols(_permute_ifgo_to_ifog(w_hh), H)
    b_p = _scale_sigmoid_cols(_permute_ifgo_to_ifog(b_ih + b_hh), H)
    wih_t = jnp.transpose(wih_p).astype(mm_dtype)
    whh_t = jnp.transpose(whh_p).astype(mm_dtype)
    bias = b_p.reshape(1, G).astype(jnp.float32)
    wfc_t = jnp.transpose(w_fc).astype(mm_dtype)
    bfc = b_fc.reshape(1, C).astype(jnp.float32)

    def _const(shape):
        return pl.BlockSpec(shape, lambda bt, c: (0, 0))

    out = pl.pallas_call(
        _lstm_kernel,
        out_shape=jax.ShapeDtypeStruct((B, C), jnp.float32),
        grid_spec=pltpu.PrefetchScalarGridSpec(
            num_scalar_prefetch=0,
            grid=(2, n_chunks),
            in_specs=[
                pl.BlockSpec((t_chunk, b_tile, D), lambda bt, c: (c, bt, 0)),
                _const((D, G)),
                _const((H, G)),
                _const((1, G)),
                _const((H, C)),
                _const((1, C)),
            ],
            out_specs=pl.BlockSpec((b_tile, C), lambda bt, c: (bt, 0)),
            scratch_shapes=[
                pltpu.VMEM((b_tile, H), jnp.float32),
                pltpu.VMEM((b_tile, H), jnp.float32),
            ],
        ),
        compiler_params=pltpu.CompilerParams(
            dimension_semantics=("parallel", "arbitrary"),
            vmem_limit_bytes=100 * 1024 * 1024,
        ),
        cost_estimate=pl.CostEstimate(
            flops=2 * T * B * (D + H) * G + 2 * B * H * C,
            transcendentals=5 * T * B * H,
            bytes_accessed=B * T * D * 2 + (D + H) * G * 2 + B * C * 4,
        ),
    )(x_tm, wih_t, whh_t, bias, wfc_t, bfc)

    return out
